# Initial kernel scaffold; baseline (speedup 1.0000x reference)
import jax
import jax.numpy as jnp
from jax.experimental import pallas as pl


def kernel(x, gate_w, gate_b, bias, shared_w1, shared_b1, shared_w2, shared_b2, expert_w1, expert_b1, expert_w2, expert_b2):
    raise NotImplementedError("write your pallas kernel here")


# speedup vs baseline: 8.4433x; 8.4433x over previous
import functools

import jax
import jax.numpy as jnp
from jax import lax
from jax.experimental import pallas as pl
from jax.experimental.pallas import tpu as pltpu
from jax.experimental.pallas import tpu_sc as plsc

H = 768
FF = 4 * H
E = 64
K = 2
S = 2048
A = S * K
T = 64
G = A // T + E - 1
GP = G + 1
NPAD = G * T
NW = 32
APW = A // NW
TPW = S // NW
CH = 32
WL = 128


def _gelu_exact(v):
    return 0.5 * v * (1.0 + lax.erf(v * 0.7071067811865476))


def _cumsum0_excl(f):
    acc = f
    d = 1
    while d < S:
        shifted = jnp.concatenate(
            [jnp.zeros((d, E), jnp.float32), acc[: S - d, :]], axis=0)
        acc = acc + shifted
        d *= 2
    return acc - f


def _route_kernel(x_ref, gw_ref, gb_ref, bias_ref, pos_ref, w16_ref, te_ref):
    x = x_ref[...]
    logits = (jnp.dot(x, gw_ref[...], preferred_element_type=jnp.float32)
              + gb_ref[...] + bias_ref[...])
    col = lax.broadcasted_iota(jnp.int32, (S, E), 1)
    m1 = jnp.max(logits, axis=1, keepdims=True)
    i1 = jnp.min(jnp.where(logits == m1, col, E), axis=1, keepdims=True)
    oh1 = col == i1
    masked = jnp.where(oh1, -jnp.inf, logits)
    m2 = jnp.max(masked, axis=1, keepdims=True)
    i2 = jnp.min(jnp.where(masked == m2, col, E), axis=1, keepdims=True)
    oh2 = col == i2
    e2 = jnp.exp(m2 - m1)
    wa = 1.0 / (1.0 + e2)
    wb = e2 * wa

    f1 = oh1.astype(jnp.float32)
    f2 = oh2.astype(jnp.float32)
    c1 = _cumsum0_excl(f1)
    c2 = _cumsum0_excl(f2)
    cnt1 = jnp.sum(f1, axis=0, keepdims=True)
    cnt = cnt1 + jnp.sum(f2, axis=0, keepdims=True)
    rank1 = jnp.sum(f1 * c1, axis=1, keepdims=True)
    rank2 = jnp.sum(f2 * (cnt1 + c2), axis=1, keepdims=True)

    cnt_i = cnt.astype(jnp.int32)
    pc = ((cnt_i + (T - 1)) // T) * T
    pcf = pc.astype(jnp.float32)
    off = pcf
    d = 1
    while d < E:
        off = off + jnp.concatenate(
            [jnp.zeros((1, d), jnp.float32), off[:, : E - d]], axis=1)
        d *= 2
    off = off - pcf

    pos1 = jnp.sum(f1 * off, axis=1, keepdims=True) + rank1
    pos2 = jnp.sum(f2 * off, axis=1, keepdims=True) + rank2
    pos_ref[0:1, :] = pos1.astype(jnp.int32).reshape(1, S)
    pos_ref[1:2, :] = pos2.astype(jnp.int32).reshape(1, S)
    w16_ref[0:S, :] = jnp.broadcast_to(wa, (S, WL))
    w16_ref[S:, :] = jnp.broadcast_to(wb, (S, WL))

    trow = lax.broadcasted_iota(jnp.int32, (GP, E), 0) * T
    offi = off.astype(jnp.int32)
    eids = lax.broadcasted_iota(jnp.int32, (GP, E), 1)
    ind = (trow >= offi) & (trow < offi + pc) & (pc > 0)
    te_raw = jnp.sum(jnp.where(ind, eids, 0), axis=1, keepdims=True)
    e_last = jnp.max(jnp.where(pc > 0, eids[0:1, :], 0))
    npad_i = jnp.sum(pc)
    te = jnp.where(trow[:, 0:1] < npad_i, te_raw, e_last)
    te_ref[...] = te.reshape(1, GP)


def _route_call(x_flat, gate_w, gate_b, bias):
    return pl.pallas_call(
        _route_kernel,
        out_shape=[
            jax.ShapeDtypeStruct((K, S), jnp.int32),
            jax.ShapeDtypeStruct((A, WL), jnp.float32),
            jax.ShapeDtypeStruct((1, GP), jnp.int32),
        ],
    )(x_flat, gate_w, gate_b.reshape(1, E), bias.reshape(1, E))


@functools.cache
def _sc_mesh():
    return plsc.VectorSubcoreMesh(core_axis_name="c", subcore_axis_name="s")


@functools.cache
def _dispatch_sc():
    @functools.partial(
        pl.kernel,
        mesh=_sc_mesh(),
        out_type=[
            jax.ShapeDtypeStruct((NPAD, H), jnp.float32),
            jax.ShapeDtypeStruct((NPAD, WL), jnp.float32),
        ],
        scratch_types=[
            pltpu.VMEM((APW,), jnp.int32),
            pltpu.VMEM((APW, H), jnp.float32),
            pltpu.VMEM((APW, WL), jnp.float32),
            pltpu.SemaphoreType.DMA,
        ],
    )
    def dispatch(x_hbm, pos_hbm, w16_hbm, xs_hbm, sw_hbm,
                 idx_v, rows_v, w16_v, sem):
        wid = lax.axis_index("s") * 2 + lax.axis_index("c")
        j0 = wid * APW
        tok0 = lax.rem(j0, S)
        pltpu.sync_copy(pos_hbm.at[pl.ds(j0, APW)], idx_v)
        pltpu.sync_copy(x_hbm.at[pl.ds(tok0, APW)], rows_v)
        pltpu.sync_copy(w16_hbm.at[pl.ds(j0, APW)], w16_v)
        pltpu.async_copy(rows_v, xs_hbm.at[idx_v], sem).wait()
        pltpu.async_copy(w16_v, sw_hbm.at[idx_v], sem).wait()

    return dispatch


def _dispatch_call(x_flat, pos_flat, w16):
    return _dispatch_sc()(x_flat, pos_flat, w16)


def _ffn_kernel(te_ref, xs_ref, sw_ref, w1_ref, b1_ref, w2_ref, b2_ref,
                out_ref):
    h = (jnp.dot(xs_ref[...], w1_ref[0], preferred_element_type=jnp.float32)
         + b1_ref[0])
    h = _gelu_exact(h)
    y = (jnp.dot(h, w2_ref[0], preferred_element_type=jnp.float32)
         + b2_ref[0])
    out_ref[...] = y * sw_ref[:, 0:1]


def _ffn_call(te, xs, sw, ew1, eb1, ew2, eb2):
    grid_spec = pltpu.PrefetchScalarGridSpec(
        num_scalar_prefetch=1,
        grid=(G,),
        in_specs=[
            pl.BlockSpec((T, H), lambda t, te_r: (t, 0)),
            pl.BlockSpec((T, WL), lambda t, te_r: (t, 0)),
            pl.BlockSpec((1, H, FF), lambda t, te_r: (te_r[0, t], 0, 0)),
            pl.BlockSpec((1, 1, FF), lambda t, te_r: (te_r[0, t], 0, 0)),
            pl.BlockSpec((1, FF, H), lambda t, te_r: (te_r[0, t], 0, 0)),
            pl.BlockSpec((1, 1, H), lambda t, te_r: (te_r[0, t], 0, 0)),
        ],
        out_specs=pl.BlockSpec((T, H), lambda t, te_r: (t, 0)),
    )
    return pl.pallas_call(
        _ffn_kernel,
        grid_spec=grid_spec,
        out_shape=jax.ShapeDtypeStruct((NPAD, H), jnp.float32),
    )(te, xs, sw, ew1, eb1.reshape(E, 1, FF), ew2, eb2.reshape(E, 1, H))


def _shared_kernel(x_ref, w1_ref, b1_ref, w2_ref, b2_ref, out_ref):
    h = (jnp.dot(x_ref[...], w1_ref[...], preferred_element_type=jnp.float32)
         + b1_ref[...])
    h = _gelu_exact(h)
    out_ref[...] = (jnp.dot(h, w2_ref[...], preferred_element_type=jnp.float32)
                    + b2_ref[...])


def _shared_call(x_flat, sw1, sb1, sw2, sb2):
    SB = 256
    return pl.pallas_call(
        _shared_kernel,
        grid=(S // SB,),
        in_specs=[
            pl.BlockSpec((SB, H), lambda i: (i, 0)),
            pl.BlockSpec((H, FF), lambda i: (0, 0)),
            pl.BlockSpec((1, FF), lambda i: (0, 0)),
            pl.BlockSpec((FF, H), lambda i: (0, 0)),
            pl.BlockSpec((1, H), lambda i: (0, 0)),
        ],
        out_specs=pl.BlockSpec((SB, H), lambda i: (i, 0)),
        out_shape=jax.ShapeDtypeStruct((S, H), jnp.float32),
    )(x_flat, sw1, sb1.reshape(1, FF), sw2, sb2.reshape(1, H))


@functools.cache
def _combine_sc():
    @functools.partial(
        pl.kernel,
        mesh=_sc_mesh(),
        out_type=jax.ShapeDtypeStruct((S, H), jnp.float32),
        scratch_types=[
            pltpu.VMEM((CH,), jnp.int32),
            pltpu.VMEM((CH,), jnp.int32),
            pltpu.VMEM((CH, H), jnp.float32),
            pltpu.VMEM((CH, H), jnp.float32),
            pltpu.VMEM((CH, H), jnp.float32),
            pltpu.SemaphoreType.DMA,
        ],
    )
    def combine(ffn_hbm, ys_hbm, pos_hbm, out_hbm,
                i0_v, i1_v, r0_v, r1_v, acc_v, sem):
        wid = lax.axis_index("s") * 2 + lax.axis_index("c")

        def chunk(c, carry):
            b = wid * TPW + c * CH
            pltpu.sync_copy(pos_hbm.at[pl.ds(b, CH)], i0_v)
            pltpu.sync_copy(pos_hbm.at[pl.ds(S + b, CH)], i1_v)
            pltpu.async_copy(ffn_hbm.at[i0_v], r0_v, sem).wait()
            pltpu.async_copy(ffn_hbm.at[i1_v], r1_v, sem).wait()
            pltpu.sync_copy(ys_hbm.at[pl.ds(b, CH)], acc_v)

            def body(q, carry2):
                i = q // (H // 16)
                sl = pl.ds((q % (H // 16)) * 16, 16)
                acc_v[i, sl] = acc_v[i, sl] + r0_v[i, sl] + r1_v[i, sl]
                return carry2

            lax.fori_loop(0, CH * (H // 16), body, 0)
            pltpu.sync_copy(acc_v, out_hbm.at[pl.ds(b, CH)])
            return carry

        lax.fori_loop(0, TPW // CH, chunk, 0)

    return combine


def _combine_call(ffn_out, ys, pos_flat):
    return _combine_sc()(ffn_out, ys, pos_flat)


def kernel(x, gate_w, gate_b, bias, shared_w1, shared_b1, shared_w2,
           shared_b2, expert_w1, expert_b1, expert_w2, expert_b2):
    x_flat = x.reshape(S, H)
    pos, w16, te = _route_call(x_flat, gate_w, gate_b, bias)
    pos_flat = pos.reshape(A)
    xs, sw = _dispatch_call(x_flat, pos_flat, w16)
    ffn_out = _ffn_call(te, xs, sw, expert_w1, expert_b1, expert_w2, expert_b2)
    ys = _shared_call(x_flat, shared_w1, shared_b1, shared_w2, shared_b2)
    out = _combine_call(ffn_out, ys, pos_flat)
    return out.reshape(x.shape)
